# baseline (speedup 1.0000x reference)
import functools

import jax
import jax.numpy as jnp
from jax import lax
from jax.experimental import pallas as pl
from jax.experimental.pallas import tpu as pltpu
from jax.experimental.pallas import tpu_sc as plsc

L = 16


def _matvec_relu_matvec(src_scr, w1t, b1, w2t, b2, d_in, d_mid, d_out, h_scr):
    n_mid = d_mid // L
    n_out = d_out // L

    def body1(jc, acc):
        vchunk = src_scr[pl.ds(jc * L, L)]
        for jl in range(L):
            bj = jnp.full((L,), vchunk[jl], jnp.float32)
            acc = tuple(acc[o] + bj * w1t[jc * L + jl, pl.ds(o * L, L)]
                        for o in range(n_mid))
        return acc

    acc0 = tuple(b1[pl.ds(o * L, L)] for o in range(n_mid))
    h = lax.fori_loop(0, d_in // L, body1, acc0)
    for o in range(n_mid):
        h_scr[pl.ds(o * L, L)] = jnp.maximum(h[o], 0.0)

    def body2(jc, acc):
        vchunk = h_scr[pl.ds(jc * L, L)]
        for jl in range(L):
            bj = jnp.full((L,), vchunk[jl], jnp.float32)
            acc = tuple(acc[o] + bj * w2t[jc * L + jl, pl.ds(o * L, L)]
                        for o in range(n_out))
        return acc

    acc1 = tuple(b2[pl.ds(o * L, L)] for o in range(n_out))
    return lax.fori_loop(0, d_mid // L, body2, acc1)


def _sc_body(x0_h, w1s_h, b1s_h, w2s_h, b2s_h, w1c_h, b1c_h, w2c_h, b2c_h,
             out_h, w1s, b1s, w2s, b2s, w1c, b1c, w2c, b2c,
             v_scr, h_scr, o_scr, *, n, latent, d_out):
    cid = lax.axis_index("c")
    sid = lax.axis_index("s")

    @pl.when(jnp.logical_and(cid == 0, sid == 0))
    def _():
        pltpu.sync_copy(x0_h, v_scr)
        pltpu.sync_copy(w1s_h, w1s)
        pltpu.sync_copy(b1s_h, b1s)
        pltpu.sync_copy(w2s_h, w2s)
        pltpu.sync_copy(b2s_h, b2s)
        pltpu.sync_copy(w1c_h, w1c)
        pltpu.sync_copy(b1c_h, b1c)
        pltpu.sync_copy(w2c_h, w2c)
        pltpu.sync_copy(b2c_h, b2c)

        n_lat = latent // L
        n_o = d_out // L

        def spread_step(_, carry):
            v = _matvec_relu_matvec(v_scr, w1s, b1s, w2s, b2s,
                                    latent, 2 * latent, latent, h_scr)
            for c in range(n_lat):
                v_scr[pl.ds(c * L, L)] = v[c]
            return carry

        lax.fori_loop(0, n - 1, spread_step, 0)

        def collect_step(ii, pos):
            @pl.when(pos != ii)
            def _():
                for c in range(n_lat):
                    v_scr[pl.ds(c * L, L)] = jnp.zeros((L,), jnp.float32)
            v = _matvec_relu_matvec(v_scr, w1c, b1c, w2c, b2c,
                                    latent, 2 * d_out, d_out, h_scr)
            for c in range(n_o):
                v_scr[pl.ds(c * L, L)] = v[c]
            return ii - 1

        pos = lax.fori_loop(1, n, collect_step, n - 1)

        for c in range(n_o):
            o_scr[pl.ds(c * L, L)] = v_scr[pl.ds(c * L, L)]

        @pl.when(pos != 0)
        def _():
            for c in range(n_o):
                o_scr[pl.ds(c * L, L)] = jnp.zeros((L,), jnp.float32)
        pltpu.sync_copy(o_scr, out_h)


def kernel(x, num_node, edge_index, W1s, b1s, W2s, b2s, W1c, b1c, W2c, b2c):
    del num_node, edge_index
    n = x.shape[0]
    latent = W2s.shape[0]
    d_out = W2c.shape[0]
    x0 = jnp.pad(x[0, :], (0, latent - x.shape[1]))

    mesh = plsc.VectorSubcoreMesh(core_axis_name="c", subcore_axis_name="s")
    body = functools.partial(_sc_body, n=n, latent=latent, d_out=d_out)
    run = pl.kernel(
        body,
        out_type=jax.ShapeDtypeStruct((d_out,), jnp.float32),
        mesh=mesh,
        scratch_types=[
            pltpu.VMEM((latent, 2 * latent), jnp.float32),
            pltpu.VMEM((2 * latent,), jnp.float32),
            pltpu.VMEM((2 * latent, latent), jnp.float32),
            pltpu.VMEM((latent,), jnp.float32),
            pltpu.VMEM((latent, 2 * d_out), jnp.float32),
            pltpu.VMEM((2 * d_out,), jnp.float32),
            pltpu.VMEM((2 * d_out, d_out), jnp.float32),
            pltpu.VMEM((d_out,), jnp.float32),
            pltpu.VMEM((latent,), jnp.float32),
            pltpu.VMEM((2 * latent,), jnp.float32),
            pltpu.VMEM((d_out,), jnp.float32),
        ],
    )
    return run(x0, W1s.T, b1s, W2s.T, b2s, W1c.T, b1c, W2c.T, b2c)


# speedup vs baseline: 1.8979x; 1.8979x over previous
import functools

import jax
import jax.numpy as jnp
from jax import lax
from jax.experimental import pallas as pl
from jax.experimental.pallas import tpu as pltpu
from jax.experimental.pallas import tpu_sc as plsc

L = 16


def _matvec_relu_matvec(src_scr, w1t, b1, w2t, b2, d_in, d_mid, d_out, h_scr):
    n_mid = d_mid // L
    n_out = d_out // L

    def body1(jc, acc):
        vchunk = src_scr[pl.ds(jc * L, L)]
        for jl in range(L):
            bj = jnp.full((L,), vchunk[jl], jnp.float32)
            acc = tuple(acc[o] + bj * w1t[jc * L + jl, pl.ds(o * L, L)]
                        for o in range(n_mid))
        return acc

    acc0 = tuple(b1[pl.ds(o * L, L)] for o in range(n_mid))
    h = lax.fori_loop(0, d_in // L, body1, acc0)
    for o in range(n_mid):
        h_scr[pl.ds(o * L, L)] = jnp.maximum(h[o], 0.0)

    def body2(jc, acc):
        vchunk = h_scr[pl.ds(jc * L, L)]
        for jl in range(L):
            bj = jnp.full((L,), vchunk[jl], jnp.float32)
            acc = tuple(acc[o] + bj * w2t[jc * L + jl, pl.ds(o * L, L)]
                        for o in range(n_out))
        return acc

    acc1 = tuple(b2[pl.ds(o * L, L)] for o in range(n_out))
    return lax.fori_loop(0, d_mid // L, body2, acc1)


def _sc_body(x0_h, w1s_h, b1s_h, w2s_h, b2s_h, w1c_h, b1c_h, w2c_h, b2c_h,
             out_h, wa, ba, wb, bb, v_scr, h_scr, o_scr, *, n, latent, d_out):
    cid = lax.axis_index("c")
    sid = lax.axis_index("s")
    n_lat = latent // L
    n_o = d_out // L

    @pl.when(jnp.logical_and(cid == 0, sid == 0))
    def _():
        pltpu.sync_copy(x0_h, v_scr)
        pltpu.sync_copy(w1s_h, wa)
        pltpu.sync_copy(b1s_h, ba)
        pltpu.sync_copy(w2s_h, wb)
        pltpu.sync_copy(b2s_h, bb)

        def spread_step(_, carry):
            v = _matvec_relu_matvec(v_scr, wa, ba, wb, bb,
                                    latent, 2 * latent, latent, h_scr)
            for c in range(n_lat):
                v_scr[pl.ds(c * L, L)] = v[c]
            return carry

        lax.fori_loop(0, n - 1, spread_step, 0)

    @pl.when(jnp.logical_and(cid == 1, sid == 0))
    def _():
        pltpu.sync_copy(w1c_h, wa)
        pltpu.sync_copy(b1c_h, ba)
        pltpu.sync_copy(w2c_h, wb)
        pltpu.sync_copy(b2c_h, bb)

        def collect_step(ii, pos):
            @pl.when(pos != ii)
            def _():
                for c in range(n_lat):
                    v_scr[pl.ds(c * L, L)] = jnp.zeros((L,), jnp.float32)

            v = _matvec_relu_matvec(v_scr, wa, ba, wb, bb,
                                    latent, 2 * d_out, d_out, h_scr)
            for c in range(n_o):
                v_scr[pl.ds(c * L, L)] = v[c]
            return ii - 1

        pos = lax.fori_loop(1, n, collect_step, n - 1)

        for c in range(n_o):
            o_scr[pl.ds(c * L, L)] = v_scr[pl.ds(c * L, L)]

        @pl.when(pos != 0)
        def _():
            for c in range(n_o):
                o_scr[pl.ds(c * L, L)] = jnp.zeros((L,), jnp.float32)

        pltpu.sync_copy(o_scr, out_h)


def kernel(x, num_node, edge_index, W1s, b1s, W2s, b2s, W1c, b1c, W2c, b2c):
    del num_node, edge_index
    n = x.shape[0]
    assert n > 2
    latent = W2s.shape[0]
    d_out = W2c.shape[0]
    x0 = jnp.pad(x[0, :], (0, latent - x.shape[1]))

    mesh = plsc.VectorSubcoreMesh(core_axis_name="c", subcore_axis_name="s")
    body = functools.partial(_sc_body, n=n, latent=latent, d_out=d_out)
    run = pl.kernel(
        body,
        out_type=jax.ShapeDtypeStruct((d_out,), jnp.float32),
        mesh=mesh,
        scratch_types=[
            pltpu.VMEM((latent, 2 * latent), jnp.float32),
            pltpu.VMEM((2 * latent,), jnp.float32),
            pltpu.VMEM((2 * latent, latent), jnp.float32),
            pltpu.VMEM((latent,), jnp.float32),
            pltpu.VMEM((latent,), jnp.float32),
            pltpu.VMEM((2 * latent,), jnp.float32),
            pltpu.VMEM((d_out,), jnp.float32),
        ],
    )
    return run(x0, W1s.T, b1s, W2s.T, b2s, W1c.T, b1c, W2c.T, b2c)
